# baseline (speedup 1.0000x reference)
import functools

import jax
import jax.numpy as jnp
from jax.experimental import pallas as pl
from jax.experimental.pallas import tpu as pltpu

_NSLOT = 2
_NSPLIT = 8


def _copies_for_step(step, out_ref, scratch_ref, sems, *, hb, n_h):
    di = step // n_h
    hi = step % n_h
    slot = step % _NSLOT
    sub = hb // _NSPLIT
    copies = []
    for j in range(_NSPLIT):
        copies.append(pltpu.make_async_copy(
            scratch_ref.at[slot, pl.ds(j * sub, sub)],
            out_ref.at[di, pl.ds(hi * hb + j * sub, sub)],
            sems.at[slot, j],
        ))
    return copies


def _pos_body(row_ref, col_ref, depth_ref, out_ref, scratch_ref, sems,
              *, hb, w, n_h, total):
    di = pl.program_id(0)
    hi = pl.program_id(1)
    step = di * n_h + hi
    slot = step % _NSLOT

    @pl.when(step >= _NSLOT)
    def _():
        for c in _copies_for_step(step - _NSLOT, out_ref, scratch_ref, sems,
                                  hb=hb, n_h=n_h):
            c.wait()

    col = col_ref[0:w, :]
    row = row_ref[...]
    depth = depth_ref[pl.ds(di, 1), :]
    zc = jnp.zeros((w, 64), jnp.float32)
    zr = jnp.zeros((hb, 64), jnp.float32)
    u = jnp.concatenate(
        [col, zc, jnp.broadcast_to(depth, (w, 64))], axis=-1)
    v = jnp.concatenate([zr, row, zr], axis=-1)
    scratch_ref[slot] = u[None, :, :] + v[:, None, :]

    for c in _copies_for_step(step, out_ref, scratch_ref, sems, hb=hb, n_h=n_h):
        c.start()

    @pl.when(step == total - 1)
    def _():
        for j in range(_NSLOT):
            for c in _copies_for_step(total - _NSLOT + j, out_ref, scratch_ref,
                                      sems, hb=hb, n_h=n_h):
                c.wait()


def kernel(scan, row_weight, col_weight, depth_weight):
    d, em, h, w = scan.shape
    hb = 32
    n_h = h // hb
    total = d * n_h
    body = functools.partial(_pos_body, hb=hb, w=w, n_h=n_h, total=total)
    out = pl.pallas_call(
        body,
        grid=(d, n_h),
        in_specs=[
            pl.BlockSpec((hb, 64), lambda di, hi: (hi, 0)),
            pl.BlockSpec((256, 64), lambda di, hi: (0, 0)),
            pl.BlockSpec((40, 64), lambda di, hi: (0, 0)),
        ],
        out_specs=pl.BlockSpec(memory_space=pltpu.MemorySpace.HBM),
        out_shape=jax.ShapeDtypeStruct((d, h, w, 192), jnp.float32),
        scratch_shapes=[
            pltpu.VMEM((_NSLOT, hb, w, 192), jnp.float32),
            pltpu.SemaphoreType.DMA((_NSLOT, _NSPLIT)),
        ],
        compiler_params=pltpu.CompilerParams(
            dimension_semantics=("arbitrary", "arbitrary")),
    )(row_weight, col_weight, depth_weight)
    return out
